# baseline (speedup 1.0000x reference)
import functools

import jax
import jax.numpy as jnp
from jax import lax
from jax.experimental import pallas as pl
from jax.experimental.pallas import tpu as pltpu
from jax.experimental.pallas import tpu_sc as plsc

B = 256
C_SIZE = 5
EMB = 128
HID = 1024
OUT = 32000
T = 50
L = 20
TXT = 250
KPAD = 64
NTILE = 10
VTILE = OUT // NTILE
FBLK = 128
SBLK = 32

_NC = 2
_NS = 16
_NW = _NC * _NS
_GN = B * C_SIZE
_GPW = _GN // _NW
_RPW = B // _NW
_ROWPAD = OUT + KPAD


def _ctx_gather_body(table_hbm, ids_hbm, ctx_hbm, gidx_v, rows_v, sem):
    wid = lax.axis_index("s") * _NC + lax.axis_index("c")
    base = wid * _GPW
    pltpu.sync_copy(ids_hbm.at[pl.ds(base, _GPW)], gidx_v)
    pltpu.async_copy(table_hbm.at[gidx_v], rows_v, sem).wait()
    pltpu.sync_copy(rows_v, ctx_hbm.at[pl.ds(base, _GPW)])


_ctx_gather = functools.partial(
    pl.kernel,
    mesh=plsc.VectorSubcoreMesh(core_axis_name="c", subcore_axis_name="s",
                                num_cores=_NC, num_subcores=_NS),
    out_type=jax.ShapeDtypeStruct((_GN, EMB), jnp.float32),
    scratch_types=[
        pltpu.VMEM((_GPW,), jnp.int32),
        pltpu.VMEM((_GPW, EMB), jnp.float32),
        pltpu.SemaphoreType.DMA,
    ],
    compiler_params=pltpu.CompilerParams(needs_layout_passes=False),
)(_ctx_gather_body)


def _bias_scatter_body(idx_hbm, val_hbm, out_hbm, row_a, row_b, iv_v, vv_v,
                       sem_a, sem_b):
    wid = lax.axis_index("s") * _NC + lax.axis_index("c")
    zero16f = jnp.zeros((16,), jnp.float32)
    bufs = (row_a, row_b)
    sems = (sem_a, sem_b)

    def _zinit(i, carry):
        row_a[pl.ds(i * 16, 16)] = zero16f
        row_b[pl.ds(i * 16, 16)] = zero16f
        return carry

    lax.fori_loop(0, _ROWPAD // 16, _zinit, 0)

    pltpu.sync_copy(idx_hbm.at[pl.ds(wid * _RPW, _RPW)], iv_v)
    pltpu.sync_copy(val_hbm.at[pl.ds(wid * _RPW, _RPW)], vv_v)

    pend = [None, None]
    for r in range(_RPW):
        bsel = r % 2
        rv = bufs[bsel]
        if pend[bsel] is not None:
            pend[bsel].wait()
            for c in range(KPAD // 16):
                ii = iv_v[r - 2, pl.ds(c * 16, 16)]
                plsc.store_scatter(rv, [ii], zero16f)
        for c in range(KPAD // 16):
            ii = iv_v[r, pl.ds(c * 16, 16)]
            xx = vv_v[r, pl.ds(c * 16, 16)]
            plsc.addupdate_scatter(rv, [ii], xx)
        pend[bsel] = pltpu.async_copy(
            rv.at[pl.ds(0, OUT)], out_hbm.at[wid * _RPW + r], sems[bsel])
    pend[0].wait()
    pend[1].wait()


_bias_scatter = functools.partial(
    pl.kernel,
    mesh=plsc.VectorSubcoreMesh(core_axis_name="c", subcore_axis_name="s",
                                num_cores=_NC, num_subcores=_NS),
    out_type=jax.ShapeDtypeStruct((B, OUT), jnp.float32),
    scratch_types=[
        pltpu.VMEM((_ROWPAD,), jnp.float32),
        pltpu.VMEM((_ROWPAD,), jnp.float32),
        pltpu.VMEM((_RPW, KPAD), jnp.int32),
        pltpu.VMEM((_RPW, KPAD), jnp.float32),
        pltpu.SemaphoreType.DMA,
        pltpu.SemaphoreType.DMA,
    ],
    compiler_params=pltpu.CompilerParams(needs_layout_passes=False),
)(_bias_scatter_body)


def _digit(n, ln, k):
    e = jnp.maximum(ln - 1 - k, 0)
    pw = jnp.where(e == 0, 1,
         jnp.where(e == 1, 10,
         jnp.where(e == 2, 100,
         jnp.where(e == 3, 1000, 10000))))
    return (n // pw) % 10


def _ndigits(n):
    return (1 + (n >= 10).astype(jnp.int32) + (n >= 100).astype(jnp.int32)
            + (n >= 1000).astype(jnp.int32) + (n >= 10000).astype(jnp.int32))


def _shift_right_sel(x, amt, lo, hi):
    S = x.shape[-1]
    out = jnp.zeros_like(x)
    amtb = jnp.broadcast_to(amt, x.shape)
    for dlt in range(lo, hi + 1):
        sh = jnp.concatenate([jnp.zeros_like(x[:, :dlt]), x[:, : S - dlt]], axis=-1)
        out = jnp.where(amtb == dlt, sh, out)
    return out


def _features_body(ps_ref, src_ref, slen_ref, val_ref, idx_ref):
    i32, f32 = jnp.int32, jnp.float32
    src = src_ref[...]
    ps = ps_ref[...]
    slen = slen_ref[...]

    l_src = _ndigits(src)
    y = l_src
    for sh in (1, 2, 4, 8, 16, 32):
        y = y + jnp.concatenate([jnp.zeros_like(y[:, :sh]), y[:, :-sh]], axis=-1)
    offs = y - l_src
    tlen = y[:, T - 1:T]

    r_dig = []
    q = src
    for _ in range(5):
        r_dig.append(q % 10)
        q = q // 10

    p3 = lax.broadcasted_iota(i32, (1, 1, TXT + 4), 2)
    acc3 = jnp.zeros((FBLK, T, TXT + 4), i32)
    for k in range(5):
        e = l_src - 1 - k
        dk = jnp.where(e == 1, r_dig[1],
             jnp.where(e == 2, r_dig[2],
             jnp.where(e == 3, r_dig[3],
             jnp.where(e == 4, r_dig[4], r_dig[0]))))
        posk = jnp.where(k < l_src, offs + k, -7)
        hit = posk[:, :, None] == p3
        acc3 = acc3 + jnp.where(hit, dk[:, :, None], 0)
    acc = jnp.sum(acc3, axis=1)

    w5 = (10000 * acc[:, 0:TXT] + 1000 * acc[:, 1:TXT + 1]
          + 100 * acc[:, 2:TXT + 2] + 10 * acc[:, 3:TXT + 3]
          + acc[:, 4:TXT + 4])

    def _p5(ln):
        return jnp.where(ln == 1, 10000,
               jnp.where(ln == 2, 1000,
               jnp.where(ln == 3, 100,
               jnp.where(ln == 4, 10, 1))))

    s3 = lax.broadcasted_iota(i32, (1, 1, TXT), 2)
    p5t = _p5(l_src)
    lo = src * p5t
    diff = w5[:, None, :] - lo[:, :, None]
    p5b = jnp.broadcast_to(p5t[:, :, None], (FBLK, T, TXT))
    endb = jnp.broadcast_to((l_src - tlen)[:, :, None], (FBLK, T, TXT))
    M = ((diff >= 0) & (diff < p5b) & (s3 <= -endb)).astype(i32)

    def match_mask(num):
        n = _ndigits(num)
        p5n = jnp.broadcast_to(_p5(n), (FBLK, TXT))
        d2 = w5 - jnp.broadcast_to(num * _p5(n), (FBLK, TXT))
        s2 = lax.broadcasted_iota(i32, (1, TXT), 1)
        fit = jnp.broadcast_to(tlen - n, (FBLK, TXT))
        return ((d2 >= 0) & (d2 < p5n) & (s2 <= fit)).astype(i32), n

    l1 = ps[:, L - 1:L]
    l0 = ps[:, L - 2:L - 1]
    ml1, n1 = match_mask(l1)
    ml0, n0 = match_mask(l0)

    ml1R = _shift_right_sel(ml1, n1, 1, 5)
    ml1R3 = jnp.broadcast_to(ml1R[:, None, :], (FBLK, T, TXT))
    found_bi = jnp.max(ml1R3 * M, axis=2)
    preR = _shift_right_sel(ml0, n0 + n1, 2, 10) * ml1R
    preR3 = jnp.broadcast_to(preR[:, None, :], (FBLK, T, TXT))
    found_tri = jnp.max(preR3 * M, axis=2)

    kk = lax.broadcasted_iota(i32, (1, T), 1)
    kk3 = lax.broadcasted_iota(i32, (1, 1, T), 2)
    eq = src[:, :, None] == src[:, None, :]
    first_occ = jnp.min(jnp.where(eq, kk3, T), axis=2)
    is_l1 = (src == jnp.broadcast_to(l1, (FBLK, T))).astype(i32)
    present1 = jnp.max(is_l1, axis=1, keepdims=True)
    kpos = jnp.min(jnp.where(is_l1 == 1, jnp.broadcast_to(kk, (FBLK, T)), T),
                   axis=1, keepdims=True)

    ok1 = (first_occ < jnp.broadcast_to(slen, (FBLK, T))).astype(i32)
    reo_or = (jnp.broadcast_to(present1, (FBLK, T))
              * (first_occ < jnp.broadcast_to(kpos, (FBLK, T))).astype(i32))
    tot = (ok1 * (1 + found_bi + found_tri + reo_or)).astype(f32)
    fm = first_occ == jnp.broadcast_to(kk, (FBLK, T))
    vals = jnp.where(fm, 0.5 * tot, 0.0)
    idx = jnp.where(fm, src, OUT + jnp.broadcast_to(kk, (FBLK, T)))

    pad_i = lax.broadcasted_iota(i32, (FBLK, KPAD - T), 1) + (OUT + T)
    val_ref[...] = jnp.concatenate(
        [vals, jnp.zeros((FBLK, KPAD - T), f32)], axis=1)
    idx_ref[...] = jnp.concatenate([idx, pad_i], axis=1)


_features = pl.pallas_call(
    _features_body,
    grid=(B // FBLK,),
    in_specs=[
        pl.BlockSpec((FBLK, L), lambda i: (i, 0)),
        pl.BlockSpec((FBLK, T), lambda i: (i, 0)),
        pl.BlockSpec((FBLK, 1), lambda i: (i, 0)),
    ],
    out_specs=[
        pl.BlockSpec((FBLK, KPAD), lambda i: (i, 0)),
        pl.BlockSpec((FBLK, KPAD), lambda i: (i, 0)),
    ],
    out_shape=[
        jax.ShapeDtypeStruct((B, KPAD), jnp.float32),
        jax.ShapeDtypeStruct((B, KPAD), jnp.int32),
    ],
)


def _nt_dot(a, b):
    return lax.dot_general(a, b, (((1,), (1,)), ((), ())),
                           preferred_element_type=jnp.float32)


def _matmul_body(ctx_ref, w1_ref, b1_ref, enc_ref, w2_ref, b2_ref,
                 w3_ref, b3_ref, out_ref, h_ref, x_ref):
    bf16 = jnp.bfloat16

    @pl.when(pl.program_id(0) == 0)
    def _():
        h = jnp.tanh(_nt_dot(ctx_ref[...], w1_ref[...]) + b1_ref[...])
        h_ref[...] = h.astype(bf16)
        x_ref[...] = enc_ref[...].astype(bf16)

    acc = (_nt_dot(h_ref[...], w2_ref[...].astype(bf16))
           + _nt_dot(x_ref[...], w3_ref[...].astype(bf16)))
    out_ref[...] = acc + b2_ref[0] + b3_ref[0]


_matmul = pl.pallas_call(
    _matmul_body,
    grid=(NTILE,),
    in_specs=[
        pl.BlockSpec((B, C_SIZE * EMB), lambda j: (0, 0)),
        pl.BlockSpec((HID, C_SIZE * EMB), lambda j: (0, 0)),
        pl.BlockSpec((1, HID), lambda j: (0, 0)),
        pl.BlockSpec((B, HID), lambda j: (0, 0)),
        pl.BlockSpec((VTILE, HID), lambda j: (j, 0)),
        pl.BlockSpec((1, 1, VTILE), lambda j: (j, 0, 0)),
        pl.BlockSpec((VTILE, HID), lambda j: (j, 0)),
        pl.BlockSpec((1, 1, VTILE), lambda j: (j, 0, 0)),
    ],
    out_specs=pl.BlockSpec((B, VTILE), lambda j: (0, j)),
    out_shape=jax.ShapeDtypeStruct((B, OUT), jnp.float32),
    scratch_shapes=[pltpu.VMEM((B, HID), jnp.bfloat16),
                    pltpu.VMEM((B, HID), jnp.bfloat16)],
    compiler_params=pltpu.CompilerParams(vmem_limit_bytes=100 * 1024 * 1024),
)


def _softmax_body(x_ref, bias_ref, o_ref):
    x = x_ref[...] + bias_ref[...]
    m = jnp.max(x, axis=1, keepdims=True)
    e = jnp.exp(x - m)
    o_ref[...] = e / jnp.sum(e, axis=1, keepdims=True)


_softmax = pl.pallas_call(
    _softmax_body,
    grid=(B // SBLK,),
    in_specs=[pl.BlockSpec((SBLK, OUT), lambda i: (i, 0)),
              pl.BlockSpec((SBLK, OUT), lambda i: (i, 0))],
    out_specs=pl.BlockSpec((SBLK, OUT), lambda i: (i, 0)),
    out_shape=jax.ShapeDtypeStruct((B, OUT), jnp.float32),
)


def kernel(encoder_output, predict_sequence, source, source_lens,
           E_table, W1, b1, W2, b2, W3, b3):
    ps = predict_sequence.astype(jnp.int32)
    src = source.astype(jnp.int32)
    slen = source_lens.astype(jnp.int32).reshape(B, 1)

    ids = ps[:, -C_SIZE:].reshape(-1)
    ctx = _ctx_gather(E_table, ids).reshape(B, C_SIZE * EMB)
    vals, idx = _features(ps, src, slen)
    bias = _bias_scatter(idx, vals)

    logits = _matmul(ctx, W1, b1.reshape(1, HID), encoder_output,
                     W2, b2.reshape(NTILE, 1, VTILE),
                     W3, b3.reshape(NTILE, 1, VTILE))
    return _softmax(logits, bias)


# speedup vs baseline: 1.0702x; 1.0702x over previous
import functools

import jax
import jax.numpy as jnp
from jax import lax
from jax.experimental import pallas as pl
from jax.experimental.pallas import tpu as pltpu
from jax.experimental.pallas import tpu_sc as plsc

B = 256
C_SIZE = 5
EMB = 128
HID = 1024
OUT = 32000
T = 50
L = 20
TXT = 250
KPAD = 64
NTILE = 25
VTILE = OUT // NTILE
FBLK = 128
SBLK = 32

_NC = 2
_NS = 16
_NW = _NC * _NS
_GN = B * C_SIZE
_GPW = _GN // _NW
_RPW = B // _NW
_ROWPAD = OUT + KPAD


def _ctx_gather_body(table_hbm, ids_hbm, ctx_hbm, gidx_v, rows_v, sem):
    wid = lax.axis_index("s") * _NC + lax.axis_index("c")
    base = wid * _GPW
    pltpu.sync_copy(ids_hbm.at[pl.ds(base, _GPW)], gidx_v)
    pltpu.async_copy(table_hbm.at[gidx_v], rows_v, sem).wait()
    pltpu.sync_copy(rows_v, ctx_hbm.at[pl.ds(base, _GPW)])


_ctx_gather = functools.partial(
    pl.kernel,
    mesh=plsc.VectorSubcoreMesh(core_axis_name="c", subcore_axis_name="s",
                                num_cores=_NC, num_subcores=_NS),
    out_type=jax.ShapeDtypeStruct((_GN, EMB), jnp.float32),
    scratch_types=[
        pltpu.VMEM((_GPW,), jnp.int32),
        pltpu.VMEM((_GPW, EMB), jnp.float32),
        pltpu.SemaphoreType.DMA,
    ],
    compiler_params=pltpu.CompilerParams(needs_layout_passes=False),
)(_ctx_gather_body)


def _bias_scatter_body(idx_hbm, val_hbm, out_hbm, row_a, row_b, iv_v, vv_v,
                       sem_a, sem_b):
    wid = lax.axis_index("s") * _NC + lax.axis_index("c")
    zero16f = jnp.zeros((16,), jnp.float32)
    bufs = (row_a, row_b)
    sems = (sem_a, sem_b)

    def _zinit(i, carry):
        row_a[pl.ds(i * 16, 16)] = zero16f
        row_b[pl.ds(i * 16, 16)] = zero16f
        return carry

    lax.fori_loop(0, _ROWPAD // 16, _zinit, 0)

    pltpu.sync_copy(idx_hbm.at[pl.ds(wid * _RPW, _RPW)], iv_v)
    pltpu.sync_copy(val_hbm.at[pl.ds(wid * _RPW, _RPW)], vv_v)

    pend = [None, None]
    for r in range(_RPW):
        bsel = r % 2
        rv = bufs[bsel]
        if pend[bsel] is not None:
            pend[bsel].wait()
            for c in range(KPAD // 16):
                ii = iv_v[r - 2, pl.ds(c * 16, 16)]
                plsc.store_scatter(rv, [ii], zero16f)
        for c in range(KPAD // 16):
            ii = iv_v[r, pl.ds(c * 16, 16)]
            xx = vv_v[r, pl.ds(c * 16, 16)]
            plsc.addupdate_scatter(rv, [ii], xx)
        pend[bsel] = pltpu.async_copy(
            rv.at[pl.ds(0, OUT)], out_hbm.at[wid * _RPW + r], sems[bsel])
    pend[0].wait()
    pend[1].wait()


_bias_scatter = functools.partial(
    pl.kernel,
    mesh=plsc.VectorSubcoreMesh(core_axis_name="c", subcore_axis_name="s",
                                num_cores=_NC, num_subcores=_NS),
    out_type=jax.ShapeDtypeStruct((B, OUT), jnp.float32),
    scratch_types=[
        pltpu.VMEM((_ROWPAD,), jnp.float32),
        pltpu.VMEM((_ROWPAD,), jnp.float32),
        pltpu.VMEM((_RPW, KPAD), jnp.int32),
        pltpu.VMEM((_RPW, KPAD), jnp.float32),
        pltpu.SemaphoreType.DMA,
        pltpu.SemaphoreType.DMA,
    ],
    compiler_params=pltpu.CompilerParams(needs_layout_passes=False),
)(_bias_scatter_body)


def _digit(n, ln, k):
    e = jnp.maximum(ln - 1 - k, 0)
    pw = jnp.where(e == 0, 1,
         jnp.where(e == 1, 10,
         jnp.where(e == 2, 100,
         jnp.where(e == 3, 1000, 10000))))
    return (n // pw) % 10


def _ndigits(n):
    return (1 + (n >= 10).astype(jnp.int32) + (n >= 100).astype(jnp.int32)
            + (n >= 1000).astype(jnp.int32) + (n >= 10000).astype(jnp.int32))


def _shift_right_sel(x, amt, lo, hi):
    S = x.shape[-1]
    out = jnp.zeros_like(x)
    amtb = jnp.broadcast_to(amt, x.shape)
    for dlt in range(lo, hi + 1):
        sh = jnp.concatenate([jnp.zeros_like(x[:, :dlt]), x[:, : S - dlt]], axis=-1)
        out = jnp.where(amtb == dlt, sh, out)
    return out


def _features_body(ps_ref, src_ref, slen_ref, val_ref, idx_ref):
    i32, f32 = jnp.int32, jnp.float32
    src = src_ref[...]
    ps = ps_ref[...]
    slen = slen_ref[...]

    l_src = _ndigits(src)
    y = l_src
    for sh in (1, 2, 4, 8, 16, 32):
        y = y + jnp.concatenate([jnp.zeros_like(y[:, :sh]), y[:, :-sh]], axis=-1)
    offs = y - l_src
    tlen = y[:, T - 1:T]

    r_dig = []
    q = src
    for _ in range(5):
        r_dig.append(q % 10)
        q = q // 10

    i16 = jnp.int16
    p3 = lax.broadcasted_iota(i16, (1, 1, TXT + 4), 2)
    acc3 = jnp.zeros((FBLK, T, TXT + 4), i16)
    for k in range(5):
        e = l_src - 1 - k
        dk = jnp.where(e == 1, r_dig[1],
             jnp.where(e == 2, r_dig[2],
             jnp.where(e == 3, r_dig[3],
             jnp.where(e == 4, r_dig[4], r_dig[0])))).astype(i16)
        posk = jnp.where(k < l_src, offs + k, -7).astype(i16)
        hit = posk[:, :, None] == p3
        acc3 = acc3 + jnp.where(hit, dk[:, :, None], i16(0))
    acc = jnp.sum(acc3.astype(i32), axis=1)

    w5 = (10000 * acc[:, 0:TXT] + 1000 * acc[:, 1:TXT + 1]
          + 100 * acc[:, 2:TXT + 2] + 10 * acc[:, 3:TXT + 3]
          + acc[:, 4:TXT + 4])

    def _p5(ln):
        return jnp.where(ln == 1, 10000,
               jnp.where(ln == 2, 1000,
               jnp.where(ln == 3, 100,
               jnp.where(ln == 4, 10, 1))))

    s3 = lax.broadcasted_iota(i32, (1, 1, TXT), 2)
    p5t = _p5(l_src)
    lo = src * p5t
    diff = w5[:, None, :] - lo[:, :, None]
    du = lax.bitcast_convert_type(diff, jnp.uint32)
    p5u = jnp.broadcast_to(p5t.astype(jnp.uint32)[:, :, None], (FBLK, T, TXT))
    endb = jnp.broadcast_to((l_src - tlen)[:, :, None], (FBLK, T, TXT))
    M = ((du < p5u) & (s3 <= -endb)).astype(i32)

    def match_mask(num):
        n = _ndigits(num)
        p5n = jnp.broadcast_to(_p5(n), (FBLK, TXT))
        d2 = w5 - jnp.broadcast_to(num * _p5(n), (FBLK, TXT))
        s2 = lax.broadcasted_iota(i32, (1, TXT), 1)
        fit = jnp.broadcast_to(tlen - n, (FBLK, TXT))
        return ((d2 >= 0) & (d2 < p5n) & (s2 <= fit)).astype(i32), n

    l1 = ps[:, L - 1:L]
    l0 = ps[:, L - 2:L - 1]
    ml1, n1 = match_mask(l1)
    ml0, n0 = match_mask(l0)

    ml1R = _shift_right_sel(ml1, n1, 1, 5)
    ml1R3 = jnp.broadcast_to(ml1R[:, None, :], (FBLK, T, TXT))
    found_bi = jnp.max(ml1R3 * M, axis=2)
    preR = _shift_right_sel(ml0, n0 + n1, 2, 10) * ml1R
    preR3 = jnp.broadcast_to(preR[:, None, :], (FBLK, T, TXT))
    found_tri = jnp.max(preR3 * M, axis=2)

    kk = lax.broadcasted_iota(i32, (1, T), 1)
    kk3 = lax.broadcasted_iota(i32, (1, 1, T), 2)
    eq = src[:, :, None] == src[:, None, :]
    first_occ = jnp.min(jnp.where(eq, kk3, T), axis=2)
    is_l1 = (src == jnp.broadcast_to(l1, (FBLK, T))).astype(i32)
    present1 = jnp.max(is_l1, axis=1, keepdims=True)
    kpos = jnp.min(jnp.where(is_l1 == 1, jnp.broadcast_to(kk, (FBLK, T)), T),
                   axis=1, keepdims=True)

    ok1 = (first_occ < jnp.broadcast_to(slen, (FBLK, T))).astype(i32)
    reo_or = (jnp.broadcast_to(present1, (FBLK, T))
              * (first_occ < jnp.broadcast_to(kpos, (FBLK, T))).astype(i32))
    tot = (ok1 * (1 + found_bi + found_tri + reo_or)).astype(f32)
    fm = first_occ == jnp.broadcast_to(kk, (FBLK, T))
    vals = jnp.where(fm, 0.5 * tot, 0.0)
    idx = jnp.where(fm, src, OUT + jnp.broadcast_to(kk, (FBLK, T)))

    pad_i = lax.broadcasted_iota(i32, (FBLK, KPAD - T), 1) + (OUT + T)
    val_ref[...] = jnp.concatenate(
        [vals, jnp.zeros((FBLK, KPAD - T), f32)], axis=1)
    idx_ref[...] = jnp.concatenate([idx, pad_i], axis=1)


_features = pl.pallas_call(
    _features_body,
    grid=(B // FBLK,),
    in_specs=[
        pl.BlockSpec((FBLK, L), lambda i: (i, 0)),
        pl.BlockSpec((FBLK, T), lambda i: (i, 0)),
        pl.BlockSpec((FBLK, 1), lambda i: (i, 0)),
    ],
    out_specs=[
        pl.BlockSpec((FBLK, KPAD), lambda i: (i, 0)),
        pl.BlockSpec((FBLK, KPAD), lambda i: (i, 0)),
    ],
    out_shape=[
        jax.ShapeDtypeStruct((B, KPAD), jnp.float32),
        jax.ShapeDtypeStruct((B, KPAD), jnp.int32),
    ],
)


def _nt_dot(a, b):
    return lax.dot_general(a, b, (((1,), (1,)), ((), ())),
                           preferred_element_type=jnp.float32)


def _matmul_body(ctx_ref, w1_ref, b1_ref, enc_ref, w2_ref, b2_ref,
                 w3_ref, b3_ref, out_ref, h_ref, x_ref):
    bf16 = jnp.bfloat16

    @pl.when(pl.program_id(0) == 0)
    def _():
        h = jnp.tanh(_nt_dot(ctx_ref[...], w1_ref[...]) + b1_ref[...])
        h_ref[...] = h.astype(bf16)
        x_ref[...] = enc_ref[...].astype(bf16)

    acc = (_nt_dot(h_ref[...], w2_ref[...].astype(bf16))
           + _nt_dot(x_ref[...], w3_ref[...].astype(bf16)))
    out_ref[...] = acc + b2_ref[0] + b3_ref[0]


_matmul = pl.pallas_call(
    _matmul_body,
    grid=(NTILE,),
    in_specs=[
        pl.BlockSpec((B, C_SIZE * EMB), lambda j: (0, 0)),
        pl.BlockSpec((HID, C_SIZE * EMB), lambda j: (0, 0)),
        pl.BlockSpec((1, HID), lambda j: (0, 0)),
        pl.BlockSpec((B, HID), lambda j: (0, 0)),
        pl.BlockSpec((VTILE, HID), lambda j: (j, 0)),
        pl.BlockSpec((1, 1, VTILE), lambda j: (j, 0, 0)),
        pl.BlockSpec((VTILE, HID), lambda j: (j, 0)),
        pl.BlockSpec((1, 1, VTILE), lambda j: (j, 0, 0)),
    ],
    out_specs=pl.BlockSpec((B, VTILE), lambda j: (0, j)),
    out_shape=jax.ShapeDtypeStruct((B, OUT), jnp.float32),
    scratch_shapes=[pltpu.VMEM((B, HID), jnp.bfloat16),
                    pltpu.VMEM((B, HID), jnp.bfloat16)],
    compiler_params=pltpu.CompilerParams(vmem_limit_bytes=100 * 1024 * 1024),
)


def _softmax_body(x_ref, bias_ref, o_ref):
    x = x_ref[...] + bias_ref[...]
    m = jnp.max(x, axis=1, keepdims=True)
    e = jnp.exp(x - m)
    o_ref[...] = e / jnp.sum(e, axis=1, keepdims=True)


_softmax = pl.pallas_call(
    _softmax_body,
    grid=(B // SBLK,),
    in_specs=[pl.BlockSpec((SBLK, OUT), lambda i: (i, 0)),
              pl.BlockSpec((SBLK, OUT), lambda i: (i, 0))],
    out_specs=pl.BlockSpec((SBLK, OUT), lambda i: (i, 0)),
    out_shape=jax.ShapeDtypeStruct((B, OUT), jnp.float32),
)


def kernel(encoder_output, predict_sequence, source, source_lens,
           E_table, W1, b1, W2, b2, W3, b3):
    ps = predict_sequence.astype(jnp.int32)
    src = source.astype(jnp.int32)
    slen = source_lens.astype(jnp.int32).reshape(B, 1)

    ids = ps[:, -C_SIZE:].reshape(-1)
    ctx = _ctx_gather(E_table, ids).reshape(B, C_SIZE * EMB)
    vals, idx = _features(ps, src, slen)
    bias = _bias_scatter(idx, vals)

    logits = _matmul(ctx, W1, b1.reshape(1, HID), encoder_output,
                     W2, b2.reshape(NTILE, 1, VTILE),
                     W3, b3.reshape(NTILE, 1, VTILE))
    return _softmax(logits, bias)


# speedup vs baseline: 1.0945x; 1.0227x over previous
import functools

import jax
import jax.numpy as jnp
from jax import lax
from jax.experimental import pallas as pl
from jax.experimental.pallas import tpu as pltpu
from jax.experimental.pallas import tpu_sc as plsc

B = 256
C_SIZE = 5
EMB = 128
HID = 1024
OUT = 32000
T = 50
L = 20
TXT = 250
KPAD = 64
NTILE = 25
VTILE = OUT // NTILE
FBLK = 128
SBLK = 32

_NC = 2
_NS = 16
_NW = _NC * _NS
_GN = B * C_SIZE
_GPW = _GN // _NW
_RPW = B // _NW
_ROWPAD = OUT + KPAD


TROW = 256


def _sc_prep_body(table_hbm, ids_hbm, src_hbm, ctx_hbm, text_hbm,
                  gidx_v, rows_v, sem, sv_v, txt_v):
    wid = lax.axis_index("s") * _NC + lax.axis_index("c")
    i32 = jnp.int32

    base = wid * _GPW
    pltpu.sync_copy(ids_hbm.at[pl.ds(base, _GPW)], gidx_v)
    pltpu.async_copy(table_hbm.at[gidx_v], rows_v, sem).wait()
    pltpu.sync_copy(rows_v, ctx_hbm.at[pl.ds(base, _GPW)])

    zero16 = jnp.zeros((16,), i32)
    lane = lax.iota(i32, 16)

    def _row_body(r, acc_):
        row = wid * _RPW + r
        for c in range(TROW // 16):
            txt_v[pl.ds(c * 16, 16)] = zero16
        pltpu.sync_copy(src_hbm.at[row], sv_v)
        carry = jnp.int32(0)
        for c in range(4):
            v = sv_v[pl.ds(c * 16, 16)]
            ln = (1 + (v >= 10).astype(i32) + (v >= 100).astype(i32)
                  + (v >= 1000).astype(i32) + (v >= 10000).astype(i32))
            ln = jnp.where(lane + c * 16 < T, ln, 0)
            cs = plsc.cumsum(ln)
            offs = cs - ln + carry
            carry = carry + jnp.sum(ln)
            r0 = v % 10
            q1 = v // 10
            r1 = q1 % 10
            q2 = q1 // 10
            r2 = q2 % 10
            q3 = q2 // 10
            r3 = q3 % 10
            r4 = q3 // 10
            for k in range(5):
                e = ln - 1 - k
                dk = jnp.where(e == 1, r1,
                     jnp.where(e == 2, r2,
                     jnp.where(e == 3, r3,
                     jnp.where(e == 4, r4, r0))))
                pos = jnp.where(k < ln, offs + k, TROW - 1)
                plsc.store_scatter(txt_v, [pos], dk)
        pltpu.sync_copy(txt_v, text_hbm.at[row])
        return acc_

    lax.fori_loop(0, _RPW, _row_body, 0)


_sc_prep = functools.partial(
    pl.kernel,
    mesh=plsc.VectorSubcoreMesh(core_axis_name="c", subcore_axis_name="s",
                                num_cores=_NC, num_subcores=_NS),
    out_type=(jax.ShapeDtypeStruct((_GN, EMB), jnp.float32),
              jax.ShapeDtypeStruct((B, TROW), jnp.int32)),
    scratch_types=[
        pltpu.VMEM((_GPW,), jnp.int32),
        pltpu.VMEM((_GPW, EMB), jnp.float32),
        pltpu.SemaphoreType.DMA,
        pltpu.VMEM((KPAD,), jnp.int32),
        pltpu.VMEM((TROW,), jnp.int32),
    ],
    compiler_params=pltpu.CompilerParams(needs_layout_passes=False),
)(_sc_prep_body)


def _bias_scatter_body(idx_hbm, val_hbm, out_hbm, row_a, row_b, iv_v, vv_v,
                       sem_a, sem_b):
    wid = lax.axis_index("s") * _NC + lax.axis_index("c")
    zero16f = jnp.zeros((16,), jnp.float32)
    bufs = (row_a, row_b)
    sems = (sem_a, sem_b)

    def _zinit(i, carry):
        row_a[pl.ds(i * 16, 16)] = zero16f
        row_b[pl.ds(i * 16, 16)] = zero16f
        return carry

    lax.fori_loop(0, _ROWPAD // 16, _zinit, 0)

    pltpu.sync_copy(idx_hbm.at[pl.ds(wid * _RPW, _RPW)], iv_v)
    pltpu.sync_copy(val_hbm.at[pl.ds(wid * _RPW, _RPW)], vv_v)

    pend = [None, None]
    for r in range(_RPW):
        bsel = r % 2
        rv = bufs[bsel]
        if pend[bsel] is not None:
            pend[bsel].wait()
            for c in range(KPAD // 16):
                ii = iv_v[r - 2, pl.ds(c * 16, 16)]
                plsc.store_scatter(rv, [ii], zero16f)
        for c in range(KPAD // 16):
            ii = iv_v[r, pl.ds(c * 16, 16)]
            xx = vv_v[r, pl.ds(c * 16, 16)]
            plsc.addupdate_scatter(rv, [ii], xx)
        pend[bsel] = pltpu.async_copy(
            rv.at[pl.ds(0, OUT)], out_hbm.at[wid * _RPW + r], sems[bsel])
    pend[0].wait()
    pend[1].wait()


_bias_scatter = functools.partial(
    pl.kernel,
    mesh=plsc.VectorSubcoreMesh(core_axis_name="c", subcore_axis_name="s",
                                num_cores=_NC, num_subcores=_NS),
    out_type=jax.ShapeDtypeStruct((B, OUT), jnp.float32),
    scratch_types=[
        pltpu.VMEM((_ROWPAD,), jnp.float32),
        pltpu.VMEM((_ROWPAD,), jnp.float32),
        pltpu.VMEM((_RPW, KPAD), jnp.int32),
        pltpu.VMEM((_RPW, KPAD), jnp.float32),
        pltpu.SemaphoreType.DMA,
        pltpu.SemaphoreType.DMA,
    ],
    compiler_params=pltpu.CompilerParams(needs_layout_passes=False),
)(_bias_scatter_body)


def _digit(n, ln, k):
    e = jnp.maximum(ln - 1 - k, 0)
    pw = jnp.where(e == 0, 1,
         jnp.where(e == 1, 10,
         jnp.where(e == 2, 100,
         jnp.where(e == 3, 1000, 10000))))
    return (n // pw) % 10


def _ndigits(n):
    return (1 + (n >= 10).astype(jnp.int32) + (n >= 100).astype(jnp.int32)
            + (n >= 1000).astype(jnp.int32) + (n >= 10000).astype(jnp.int32))


def _shift_right_sel(x, amt, lo, hi):
    S = x.shape[-1]
    out = jnp.zeros_like(x)
    amtb = jnp.broadcast_to(amt, x.shape)
    for dlt in range(lo, hi + 1):
        sh = jnp.concatenate([jnp.zeros_like(x[:, :dlt]), x[:, : S - dlt]], axis=-1)
        out = jnp.where(amtb == dlt, sh, out)
    return out


def _features_body(ps_ref, src_ref, slen_ref, text_ref, val_ref, idx_ref):
    i32, f32 = jnp.int32, jnp.float32
    src = src_ref[...]
    ps = ps_ref[...]
    slen = slen_ref[...]
    acc = text_ref[...]

    l_src = _ndigits(src)
    tlen = jnp.sum(l_src, axis=1, keepdims=True)

    w5 = (10000 * acc[:, 0:TXT] + 1000 * acc[:, 1:TXT + 1]
          + 100 * acc[:, 2:TXT + 2] + 10 * acc[:, 3:TXT + 3]
          + acc[:, 4:TXT + 4])

    def _p5(ln):
        return jnp.where(ln == 1, 10000,
               jnp.where(ln == 2, 1000,
               jnp.where(ln == 3, 100,
               jnp.where(ln == 4, 10, 1))))

    s3 = lax.broadcasted_iota(i32, (1, 1, TXT), 2)
    p5t = _p5(l_src)
    lo = src * p5t
    diff = w5[:, None, :] - lo[:, :, None]
    du = lax.bitcast_convert_type(diff, jnp.uint32)
    p5u = jnp.broadcast_to(p5t.astype(jnp.uint32)[:, :, None], (FBLK, T, TXT))
    endb = jnp.broadcast_to((l_src - tlen)[:, :, None], (FBLK, T, TXT))
    M = ((du < p5u) & (s3 <= -endb)).astype(i32)

    def match_mask(num):
        n = _ndigits(num)
        p5n = jnp.broadcast_to(_p5(n), (FBLK, TXT))
        d2 = w5 - jnp.broadcast_to(num * _p5(n), (FBLK, TXT))
        s2 = lax.broadcasted_iota(i32, (1, TXT), 1)
        fit = jnp.broadcast_to(tlen - n, (FBLK, TXT))
        return ((d2 >= 0) & (d2 < p5n) & (s2 <= fit)).astype(i32), n

    l1 = ps[:, L - 1:L]
    l0 = ps[:, L - 2:L - 1]
    ml1, n1 = match_mask(l1)
    ml0, n0 = match_mask(l0)

    ml1R = _shift_right_sel(ml1, n1, 1, 5)
    ml1R3 = jnp.broadcast_to(ml1R[:, None, :], (FBLK, T, TXT))
    found_bi = jnp.max(ml1R3 * M, axis=2)
    preR = _shift_right_sel(ml0, n0 + n1, 2, 10) * ml1R
    preR3 = jnp.broadcast_to(preR[:, None, :], (FBLK, T, TXT))
    found_tri = jnp.max(preR3 * M, axis=2)

    kk = lax.broadcasted_iota(i32, (1, T), 1)
    kk3 = lax.broadcasted_iota(i32, (1, 1, T), 2)
    eq = src[:, :, None] == src[:, None, :]
    first_occ = jnp.min(jnp.where(eq, kk3, T), axis=2)
    is_l1 = (src == jnp.broadcast_to(l1, (FBLK, T))).astype(i32)
    present1 = jnp.max(is_l1, axis=1, keepdims=True)
    kpos = jnp.min(jnp.where(is_l1 == 1, jnp.broadcast_to(kk, (FBLK, T)), T),
                   axis=1, keepdims=True)

    ok1 = (first_occ < jnp.broadcast_to(slen, (FBLK, T))).astype(i32)
    reo_or = (jnp.broadcast_to(present1, (FBLK, T))
              * (first_occ < jnp.broadcast_to(kpos, (FBLK, T))).astype(i32))
    tot = (ok1 * (1 + found_bi + found_tri + reo_or)).astype(f32)
    fm = first_occ == jnp.broadcast_to(kk, (FBLK, T))
    vals = jnp.where(fm, 0.5 * tot, 0.0)
    idx = jnp.where(fm, src, OUT + jnp.broadcast_to(kk, (FBLK, T)))

    pad_i = lax.broadcasted_iota(i32, (FBLK, KPAD - T), 1) + (OUT + T)
    val_ref[...] = jnp.concatenate(
        [vals, jnp.zeros((FBLK, KPAD - T), f32)], axis=1)
    idx_ref[...] = jnp.concatenate([idx, pad_i], axis=1)


_features = pl.pallas_call(
    _features_body,
    grid=(B // FBLK,),
    in_specs=[
        pl.BlockSpec((FBLK, L), lambda i: (i, 0)),
        pl.BlockSpec((FBLK, T), lambda i: (i, 0)),
        pl.BlockSpec((FBLK, 1), lambda i: (i, 0)),
        pl.BlockSpec((FBLK, TROW), lambda i: (i, 0)),
    ],
    out_specs=[
        pl.BlockSpec((FBLK, KPAD), lambda i: (i, 0)),
        pl.BlockSpec((FBLK, KPAD), lambda i: (i, 0)),
    ],
    out_shape=[
        jax.ShapeDtypeStruct((B, KPAD), jnp.float32),
        jax.ShapeDtypeStruct((B, KPAD), jnp.int32),
    ],
)


def _nt_dot(a, b):
    return lax.dot_general(a, b, (((1,), (1,)), ((), ())),
                           preferred_element_type=jnp.float32)


def _matmul_body(ctx_ref, w1_ref, b1_ref, enc_ref, w2_ref, b2_ref,
                 w3_ref, b3_ref, out_ref, h_ref, x_ref):
    bf16 = jnp.bfloat16

    @pl.when(pl.program_id(0) == 0)
    def _():
        h = jnp.tanh(_nt_dot(ctx_ref[...], w1_ref[...]) + b1_ref[...])
        h_ref[...] = h.astype(bf16)
        x_ref[...] = enc_ref[...].astype(bf16)

    acc = (_nt_dot(h_ref[...], w2_ref[...].astype(bf16))
           + _nt_dot(x_ref[...], w3_ref[...].astype(bf16)))
    out_ref[...] = acc + b2_ref[0] + b3_ref[0]


_matmul = pl.pallas_call(
    _matmul_body,
    grid=(NTILE,),
    in_specs=[
        pl.BlockSpec((B, C_SIZE * EMB), lambda j: (0, 0)),
        pl.BlockSpec((HID, C_SIZE * EMB), lambda j: (0, 0)),
        pl.BlockSpec((1, HID), lambda j: (0, 0)),
        pl.BlockSpec((B, HID), lambda j: (0, 0)),
        pl.BlockSpec((VTILE, HID), lambda j: (j, 0)),
        pl.BlockSpec((1, 1, VTILE), lambda j: (j, 0, 0)),
        pl.BlockSpec((VTILE, HID), lambda j: (j, 0)),
        pl.BlockSpec((1, 1, VTILE), lambda j: (j, 0, 0)),
    ],
    out_specs=pl.BlockSpec((B, VTILE), lambda j: (0, j)),
    out_shape=jax.ShapeDtypeStruct((B, OUT), jnp.float32),
    scratch_shapes=[pltpu.VMEM((B, HID), jnp.bfloat16),
                    pltpu.VMEM((B, HID), jnp.bfloat16)],
    compiler_params=pltpu.CompilerParams(vmem_limit_bytes=100 * 1024 * 1024),
)


def _softmax_body(x_ref, bias_ref, o_ref):
    x = x_ref[...] + bias_ref[...]
    m = jnp.max(x, axis=1, keepdims=True)
    e = jnp.exp(x - m)
    o_ref[...] = e / jnp.sum(e, axis=1, keepdims=True)


_softmax = pl.pallas_call(
    _softmax_body,
    grid=(B // SBLK,),
    in_specs=[pl.BlockSpec((SBLK, OUT), lambda i: (i, 0)),
              pl.BlockSpec((SBLK, OUT), lambda i: (i, 0))],
    out_specs=pl.BlockSpec((SBLK, OUT), lambda i: (i, 0)),
    out_shape=jax.ShapeDtypeStruct((B, OUT), jnp.float32),
)


def kernel(encoder_output, predict_sequence, source, source_lens,
           E_table, W1, b1, W2, b2, W3, b3):
    ps = predict_sequence.astype(jnp.int32)
    src = source.astype(jnp.int32)
    slen = source_lens.astype(jnp.int32).reshape(B, 1)

    ids = ps[:, -C_SIZE:].reshape(-1)
    src64 = jnp.concatenate([src, jnp.zeros((B, KPAD - T), jnp.int32)], axis=1)
    ctx_rows, text = _sc_prep(E_table, ids, src64)
    ctx = ctx_rows.reshape(B, C_SIZE * EMB)
    vals, idx = _features(ps, src, slen, text)
    bias = _bias_scatter(idx, vals)

    logits = _matmul(ctx, W1, b1.reshape(1, HID), encoder_output,
                     W2, b2.reshape(NTILE, 1, VTILE),
                     W3, b3.reshape(NTILE, 1, VTILE))
    return _softmax(logits, bias)


# speedup vs baseline: 1.0947x; 1.0002x over previous
import functools

import jax
import jax.numpy as jnp
from jax import lax
from jax.experimental import pallas as pl
from jax.experimental.pallas import tpu as pltpu
from jax.experimental.pallas import tpu_sc as plsc

B = 256
C_SIZE = 5
EMB = 128
HID = 1024
OUT = 32000
T = 50
L = 20
TXT = 250
KPAD = 64
NTILE = 25
VTILE = OUT // NTILE
FBLK = 128
SBLK = 32

_NC = 2
_NS = 16
_NW = _NC * _NS
_GN = B * C_SIZE
_GPW = _GN // _NW
_RPW = B // _NW
_ROWPAD = OUT + KPAD


TROW = 256


def _sc_prep_body(table_hbm, ids_hbm, src_hbm, ctx_hbm, text_hbm,
                  gidx_v, rows_v, sem, sv_v, txt_v):
    wid = lax.axis_index("s") * _NC + lax.axis_index("c")
    i32 = jnp.int32

    base = wid * _GPW
    pltpu.sync_copy(ids_hbm.at[pl.ds(base, _GPW)], gidx_v)
    pltpu.async_copy(table_hbm.at[gidx_v], rows_v, sem).wait()
    pltpu.sync_copy(rows_v, ctx_hbm.at[pl.ds(base, _GPW)])

    zero16 = jnp.zeros((16,), i32)
    lane = lax.iota(i32, 16)

    def _row_body(r, acc_):
        row = wid * _RPW + r
        for c in range(TROW // 16):
            txt_v[pl.ds(c * 16, 16)] = zero16
        pltpu.sync_copy(src_hbm.at[row], sv_v)
        carry = jnp.int32(0)
        for c in range(4):
            v = sv_v[pl.ds(c * 16, 16)]
            ln = (1 + (v >= 10).astype(i32) + (v >= 100).astype(i32)
                  + (v >= 1000).astype(i32) + (v >= 10000).astype(i32))
            ln = jnp.where(lane + c * 16 < T, ln, 0)
            cs = plsc.cumsum(ln)
            offs = cs - ln + carry
            carry = carry + jnp.sum(ln)
            r0 = v % 10
            q1 = v // 10
            r1 = q1 % 10
            q2 = q1 // 10
            r2 = q2 % 10
            q3 = q2 // 10
            r3 = q3 % 10
            r4 = q3 // 10
            for k in range(5):
                e = ln - 1 - k
                dk = jnp.where(e == 1, r1,
                     jnp.where(e == 2, r2,
                     jnp.where(e == 3, r3,
                     jnp.where(e == 4, r4, r0))))
                pos = jnp.where(k < ln, offs + k, TROW - 1)
                plsc.store_scatter(txt_v, [pos], dk)
        pltpu.sync_copy(txt_v, text_hbm.at[row])
        return acc_

    lax.fori_loop(0, _RPW, _row_body, 0)


_sc_prep = functools.partial(
    pl.kernel,
    mesh=plsc.VectorSubcoreMesh(core_axis_name="c", subcore_axis_name="s",
                                num_cores=_NC, num_subcores=_NS),
    out_type=(jax.ShapeDtypeStruct((_GN, EMB), jnp.float32),
              jax.ShapeDtypeStruct((B, TROW), jnp.int32)),
    scratch_types=[
        pltpu.VMEM((_GPW,), jnp.int32),
        pltpu.VMEM((_GPW, EMB), jnp.float32),
        pltpu.SemaphoreType.DMA,
        pltpu.VMEM((KPAD,), jnp.int32),
        pltpu.VMEM((TROW,), jnp.int32),
    ],
    compiler_params=pltpu.CompilerParams(needs_layout_passes=False),
)(_sc_prep_body)


def _bias_scatter_body(idx_hbm, val_hbm, out_hbm, row_a, row_b, iv_v, vv_v,
                       sem_a, sem_b):
    wid = lax.axis_index("s") * _NC + lax.axis_index("c")
    zero16f = jnp.zeros((16,), jnp.float32)
    bufs = (row_a, row_b)
    sems = (sem_a, sem_b)

    def _zinit(i, carry):
        row_a[pl.ds(i * 16, 16)] = zero16f
        row_b[pl.ds(i * 16, 16)] = zero16f
        return carry

    lax.fori_loop(0, _ROWPAD // 16, _zinit, 0)

    pltpu.sync_copy(idx_hbm.at[pl.ds(wid * _RPW, _RPW)], iv_v)
    pltpu.sync_copy(val_hbm.at[pl.ds(wid * _RPW, _RPW)], vv_v)

    pend = [None, None]
    for r in range(_RPW):
        bsel = r % 2
        rv = bufs[bsel]
        if pend[bsel] is not None:
            pend[bsel].wait()
            for c in range(KPAD // 16):
                ii = iv_v[r - 2, pl.ds(c * 16, 16)]
                plsc.store_scatter(rv, [ii], zero16f)
        for c in range(KPAD // 16):
            ii = iv_v[r, pl.ds(c * 16, 16)]
            xx = vv_v[r, pl.ds(c * 16, 16)]
            plsc.addupdate_scatter(rv, [ii], xx)
        pend[bsel] = pltpu.async_copy(
            rv.at[pl.ds(0, OUT)], out_hbm.at[wid * _RPW + r], sems[bsel])
    pend[0].wait()
    pend[1].wait()


_bias_scatter = functools.partial(
    pl.kernel,
    mesh=plsc.VectorSubcoreMesh(core_axis_name="c", subcore_axis_name="s",
                                num_cores=_NC, num_subcores=_NS),
    out_type=jax.ShapeDtypeStruct((B, OUT), jnp.float32),
    scratch_types=[
        pltpu.VMEM((_ROWPAD,), jnp.float32),
        pltpu.VMEM((_ROWPAD,), jnp.float32),
        pltpu.VMEM((_RPW, KPAD), jnp.int32),
        pltpu.VMEM((_RPW, KPAD), jnp.float32),
        pltpu.SemaphoreType.DMA,
        pltpu.SemaphoreType.DMA,
    ],
    compiler_params=pltpu.CompilerParams(needs_layout_passes=False),
)(_bias_scatter_body)


def _ndigits(n):
    return (1 + (n >= 10).astype(jnp.int32) + (n >= 100).astype(jnp.int32)
            + (n >= 1000).astype(jnp.int32) + (n >= 10000).astype(jnp.int32))


def _shift_right_sel(x, amt, lo, hi):
    S = x.shape[-1]
    out = jnp.zeros_like(x)
    amtb = jnp.broadcast_to(amt, x.shape)
    for dlt in range(lo, hi + 1):
        sh = jnp.concatenate([jnp.zeros_like(x[:, :dlt]), x[:, : S - dlt]], axis=-1)
        out = jnp.where(amtb == dlt, sh, out)
    return out


def _features_body(ps_ref, src_ref, slen_ref, text_ref, val_ref, idx_ref):
    i32, f32 = jnp.int32, jnp.float32
    src = src_ref[...]
    ps = ps_ref[...]
    slen = slen_ref[...]
    acc = text_ref[...]

    l_src = _ndigits(src)
    tlen = jnp.sum(l_src, axis=1, keepdims=True)

    w5 = (10000 * acc[:, 0:TXT] + 1000 * acc[:, 1:TXT + 1]
          + 100 * acc[:, 2:TXT + 2] + 10 * acc[:, 3:TXT + 3]
          + acc[:, 4:TXT + 4])

    def _p5(ln):
        return jnp.where(ln == 1, 10000,
               jnp.where(ln == 2, 1000,
               jnp.where(ln == 3, 100,
               jnp.where(ln == 4, 10, 1))))

    s3 = lax.broadcasted_iota(i32, (1, 1, TXT), 2)
    p5t = _p5(l_src)
    lo = src * p5t
    diff = w5[:, None, :] - lo[:, :, None]
    du = lax.bitcast_convert_type(diff, jnp.uint32)
    p5u = jnp.broadcast_to(p5t.astype(jnp.uint32)[:, :, None], (FBLK, T, TXT))
    endb = jnp.broadcast_to((l_src - tlen)[:, :, None], (FBLK, T, TXT))
    M = ((du < p5u) & (s3 <= -endb)).astype(i32)

    def match_mask(num):
        n = _ndigits(num)
        p5n = jnp.broadcast_to(_p5(n), (FBLK, TXT))
        d2 = w5 - jnp.broadcast_to(num * _p5(n), (FBLK, TXT))
        s2 = lax.broadcasted_iota(i32, (1, TXT), 1)
        fit = jnp.broadcast_to(tlen - n, (FBLK, TXT))
        return ((d2 >= 0) & (d2 < p5n) & (s2 <= fit)).astype(i32), n

    l1 = ps[:, L - 1:L]
    l0 = ps[:, L - 2:L - 1]
    ml1, n1 = match_mask(l1)
    ml0, n0 = match_mask(l0)

    ml1R = _shift_right_sel(ml1, n1, 1, 5)
    ml1R3 = jnp.broadcast_to(ml1R[:, None, :], (FBLK, T, TXT))
    found_bi = jnp.max(ml1R3 * M, axis=2)
    preR = _shift_right_sel(ml0, n0 + n1, 2, 10) * ml1R
    preR3 = jnp.broadcast_to(preR[:, None, :], (FBLK, T, TXT))
    found_tri = jnp.max(preR3 * M, axis=2)

    kk = lax.broadcasted_iota(i32, (1, T), 1)
    kk3 = lax.broadcasted_iota(i32, (1, 1, T), 2)
    eq = src[:, :, None] == src[:, None, :]
    first_occ = jnp.min(jnp.where(eq, kk3, T), axis=2)
    is_l1 = (src == jnp.broadcast_to(l1, (FBLK, T))).astype(i32)
    present1 = jnp.max(is_l1, axis=1, keepdims=True)
    kpos = jnp.min(jnp.where(is_l1 == 1, jnp.broadcast_to(kk, (FBLK, T)), T),
                   axis=1, keepdims=True)

    ok1 = (first_occ < jnp.broadcast_to(slen, (FBLK, T))).astype(i32)
    reo_or = (jnp.broadcast_to(present1, (FBLK, T))
              * (first_occ < jnp.broadcast_to(kpos, (FBLK, T))).astype(i32))
    tot = (ok1 * (1 + found_bi + found_tri + reo_or)).astype(f32)
    fm = first_occ == jnp.broadcast_to(kk, (FBLK, T))
    vals = jnp.where(fm, 0.5 * tot, 0.0)
    idx = jnp.where(fm, src, OUT + jnp.broadcast_to(kk, (FBLK, T)))

    pad_i = lax.broadcasted_iota(i32, (FBLK, KPAD - T), 1) + (OUT + T)
    val_ref[...] = jnp.concatenate(
        [vals, jnp.zeros((FBLK, KPAD - T), f32)], axis=1)
    idx_ref[...] = jnp.concatenate([idx, pad_i], axis=1)


_features = pl.pallas_call(
    _features_body,
    grid=(B // FBLK,),
    in_specs=[
        pl.BlockSpec((FBLK, L), lambda i: (i, 0)),
        pl.BlockSpec((FBLK, T), lambda i: (i, 0)),
        pl.BlockSpec((FBLK, 1), lambda i: (i, 0)),
        pl.BlockSpec((FBLK, TROW), lambda i: (i, 0)),
    ],
    out_specs=[
        pl.BlockSpec((FBLK, KPAD), lambda i: (i, 0)),
        pl.BlockSpec((FBLK, KPAD), lambda i: (i, 0)),
    ],
    out_shape=[
        jax.ShapeDtypeStruct((B, KPAD), jnp.float32),
        jax.ShapeDtypeStruct((B, KPAD), jnp.int32),
    ],
)


def _nt_dot(a, b):
    return lax.dot_general(a, b, (((1,), (1,)), ((), ())),
                           preferred_element_type=jnp.float32)


def _matmul_body(ctx_ref, w1_ref, b1_ref, enc_ref, w2_ref, b2_ref,
                 w3_ref, b3_ref, out_ref, h_ref, x_ref):
    bf16 = jnp.bfloat16

    @pl.when(pl.program_id(0) == 0)
    def _():
        h = jnp.tanh(_nt_dot(ctx_ref[...], w1_ref[...]) + b1_ref[...])
        h_ref[...] = h.astype(bf16)
        x_ref[...] = enc_ref[...].astype(bf16)

    acc = (_nt_dot(h_ref[...], w2_ref[...].astype(bf16))
           + _nt_dot(x_ref[...], w3_ref[...].astype(bf16)))
    out_ref[...] = acc + b2_ref[0] + b3_ref[0]


_matmul = pl.pallas_call(
    _matmul_body,
    grid=(NTILE,),
    in_specs=[
        pl.BlockSpec((B, C_SIZE * EMB), lambda j: (0, 0)),
        pl.BlockSpec((HID, C_SIZE * EMB), lambda j: (0, 0)),
        pl.BlockSpec((1, HID), lambda j: (0, 0)),
        pl.BlockSpec((B, HID), lambda j: (0, 0)),
        pl.BlockSpec((VTILE, HID), lambda j: (j, 0)),
        pl.BlockSpec((1, 1, VTILE), lambda j: (j, 0, 0)),
        pl.BlockSpec((VTILE, HID), lambda j: (j, 0)),
        pl.BlockSpec((1, 1, VTILE), lambda j: (j, 0, 0)),
    ],
    out_specs=pl.BlockSpec((B, VTILE), lambda j: (0, j)),
    out_shape=jax.ShapeDtypeStruct((B, OUT), jnp.float32),
    scratch_shapes=[pltpu.VMEM((B, HID), jnp.bfloat16),
                    pltpu.VMEM((B, HID), jnp.bfloat16)],
    compiler_params=pltpu.CompilerParams(vmem_limit_bytes=100 * 1024 * 1024),
)


def _softmax_body(x_ref, bias_ref, o_ref):
    x = x_ref[...] + bias_ref[...]
    m = jnp.max(x, axis=1, keepdims=True)
    e = jnp.exp(x - m)
    o_ref[...] = e / jnp.sum(e, axis=1, keepdims=True)


_softmax = pl.pallas_call(
    _softmax_body,
    grid=(B // SBLK,),
    in_specs=[pl.BlockSpec((SBLK, OUT), lambda i: (i, 0)),
              pl.BlockSpec((SBLK, OUT), lambda i: (i, 0))],
    out_specs=pl.BlockSpec((SBLK, OUT), lambda i: (i, 0)),
    out_shape=jax.ShapeDtypeStruct((B, OUT), jnp.float32),
)


def kernel(encoder_output, predict_sequence, source, source_lens,
           E_table, W1, b1, W2, b2, W3, b3):
    ps = predict_sequence.astype(jnp.int32)
    src = source.astype(jnp.int32)
    slen = source_lens.astype(jnp.int32).reshape(B, 1)

    ids = ps[:, -C_SIZE:].reshape(-1)
    src64 = jnp.concatenate([src, jnp.zeros((B, KPAD - T), jnp.int32)], axis=1)
    ctx_rows, text = _sc_prep(E_table, ids, src64)
    ctx = ctx_rows.reshape(B, C_SIZE * EMB)
    vals, idx = _features(ps, src, slen, text)
    bias = _bias_scatter(idx, vals)

    logits = _matmul(ctx, W1, b1.reshape(1, HID), encoder_output,
                     W2, b2.reshape(NTILE, 1, VTILE),
                     W3, b3.reshape(NTILE, 1, VTILE))
    return _softmax(logits, bias)
